# baseline (speedup 1.0000x reference)
import functools

import jax
import jax.numpy as jnp
from jax import lax
from jax.experimental import pallas as pl
from jax.experimental.pallas import tpu as pltpu
from jax.experimental.pallas import tpu_sc as plsc

N_NODES = 50000
SUBTOK = 20
EMB = 128

NC = 2
NS = 16
UNIT = 392
N_UNITS = 128
N_PAD = N_UNITS * UNIT
U_CORE0 = 3
U_CORE1 = 5
IDX_UNIT = UNIT * SUBTOK

_mesh = plsc.VectorSubcoreMesh(core_axis_name="c", subcore_axis_name="s")


@functools.partial(
    pl.kernel,
    out_type=jax.ShapeDtypeStruct((N_PAD, EMB), jnp.float32),
    mesh=_mesh,
    scratch_types=[
        pltpu.VMEM((IDX_UNIT,), jnp.int32),
        pltpu.VMEM((UNIT,), jnp.int32),
        pltpu.VMEM((UNIT, EMB), jnp.float32),
        pltpu.SemaphoreType.DMA,
        pltpu.SemaphoreType.DMA,
    ],
)
def _node_embedding_sc(tokens_hbm, nodes_hbm, token_table, node_table,
                       out_hbm, tok_idx_v, node_idx_v, acc_v,
                       sem_add, sem_init):
    cid = lax.axis_index("c")
    sid = lax.axis_index("s")
    n_units = lax.select(cid == 0, U_CORE0, U_CORE1)
    unit0 = lax.select(cid == 0, sid * U_CORE0,
                       NS * U_CORE0 + sid * U_CORE1)

    def unit_body(u, _):
        base = (unit0 + u) * UNIT
        pltpu.sync_copy(tokens_hbm.at[pl.ds(base * SUBTOK, IDX_UNIT)],
                        tok_idx_v)
        pltpu.sync_copy(nodes_hbm.at[pl.ds(base, UNIT)], node_idx_v)
        pltpu.async_copy(node_table.at[node_idx_v], acc_v, sem_init).wait()

        def sub_body(j, _):
            pltpu.async_copy(
                token_table.at[tok_idx_v.at[pl.ds(j * UNIT, UNIT)]],
                acc_v, sem_add, add=True)
            return 0

        lax.fori_loop(0, SUBTOK, sub_body, 0)

        def drain_body(j, _):
            pltpu.make_async_copy(
                token_table.at[pl.ds(0, UNIT)], acc_v, sem_add).wait()
            return 0

        lax.fori_loop(0, SUBTOK, drain_body, 0)
        pltpu.sync_copy(acc_v, out_hbm.at[pl.ds(base, UNIT)])
        return 0

    lax.fori_loop(0, n_units, unit_body, 0)


def kernel(tokens, nodes, token_table, node_table):
    tokens = tokens.astype(jnp.int32)
    nodes = nodes.astype(jnp.int32)
    tokens_p = jnp.zeros((N_PAD, SUBTOK), jnp.int32).at[:N_NODES].set(tokens)
    nodes_p = jnp.zeros((N_PAD,), jnp.int32).at[:N_NODES].set(nodes)
    tokens_flat = (tokens_p.reshape(N_UNITS, UNIT, SUBTOK)
                   .transpose(0, 2, 1)
                   .reshape(N_PAD * SUBTOK))
    out = _node_embedding_sc(tokens_flat, nodes_p, token_table, node_table)
    return out[:N_NODES]
